# baseline (speedup 1.0000x reference)
import functools

import jax
import jax.numpy as jnp
from jax import lax
from jax.experimental import pallas as pl
from jax.experimental.pallas import tpu as pltpu
from jax.experimental.pallas import tpu_sc as plsc

N_NODES = 10000
N_EDGES = 320000
D_NODE = 128
D_EDGE = 16
EMBED = 32
HIDDEN = 128

_K = 5
_S = N_EDGES // _K


_RB = 2000


def _proj_body(nf_ref, w1_ref, w2_ref, p1_ref, p2_ref):
    x = nf_ref[...]
    p1_ref[...] = jnp.dot(x, w1_ref[...], preferred_element_type=jnp.float32)
    p2_ref[...] = jnp.dot(x, w2_ref[...], preferred_element_type=jnp.float32)


def _node_proj(node_feat, w1, w2):
    n_blocks = N_NODES // _RB
    return pl.pallas_call(
        _proj_body,
        grid=(n_blocks,),
        in_specs=[
            pl.BlockSpec((_RB, D_NODE), lambda i: (i, 0)),
            pl.BlockSpec((D_NODE, HIDDEN), lambda i: (0, 0)),
            pl.BlockSpec((D_NODE, HIDDEN), lambda i: (0, 0)),
        ],
        out_specs=[
            pl.BlockSpec((_RB, HIDDEN), lambda i: (i, 0)),
            pl.BlockSpec((_RB, HIDDEN), lambda i: (i, 0)),
        ],
        out_shape=[
            jax.ShapeDtypeStruct((N_NODES, HIDDEN), jnp.float32),
            jax.ShapeDtypeStruct((N_NODES, HIDDEN), jnp.float32),
        ],
    )(node_feat, w1, w2)


_NC = 2
_NS = 16
_NW = _NC * _NS
_BPW = _S // _NW
_C = 40
_NCHUNK = _BPW // _C
_NBUF = 12


def _accum_rows(rows_a, rows_b):
    def row_body(r, carry):
        for c in range(HIDDEN // 16):
            sl = pl.ds(c * 16, 16)
            plsc.addupdate(rows_a.at[r, sl], rows_b[r, sl])
        return carry
    lax.fori_loop(0, _C, row_body, 0)


def _gather_body(kslice, p1_hbm, p2_hbm, ei_hbm, g_hbm,
                 idx1, idx2, rows_a, rows_b, gsems, wsem):
    wid = lax.axis_index("s") * _NC + lax.axis_index("c")
    base = pl.multiple_of(kslice * _S + wid * _BPW, 8)
    obase0 = pl.multiple_of(wid * _BPW, 8)
    pltpu.sync_copy(ei_hbm.at[pl.ds(base, _BPW)], idx1)
    pltpu.sync_copy(ei_hbm.at[pl.ds(N_EDGES + base, _BPW)], idx2)

    def fire(off, s):
        cp1 = pltpu.async_copy(p1_hbm.at[idx1.at[pl.ds(off, _C)]],
                               rows_a[s], gsems[s])
        cp2 = pltpu.async_copy(p2_hbm.at[idx2.at[pl.ds(off, _C)]],
                               rows_b[s], gsems[s])
        return cp1, cp2

    def drain(off, s, cps):
        cps[0].wait()
        cps[1].wait()
        _accum_rows(rows_a[s], rows_b[s])
        obase = pl.multiple_of(obase0 + off, 8)
        return pltpu.async_copy(rows_a[s], g_hbm.at[pl.ds(obase, _C)], wsem)

    def body(pi, carry):
        offs = [pl.multiple_of(pi * (_NBUF * _C) + s * _C, 8)
                for s in range(_NBUF)]
        cps = [fire(offs[s], s) for s in range(_NBUF)]
        ws = [drain(offs[s], s, cps[s]) for s in range(_NBUF)]
        for w in ws:
            w.wait()
        return carry

    lax.fori_loop(0, _NCHUNK // _NBUF, body, 0)
    for t in range(_NCHUNK - (_NCHUNK // _NBUF) * _NBUF):
        off = pl.multiple_of(((_NCHUNK // _NBUF) * _NBUF + t) * _C, 8)
        drain(off, t, fire(off, t)).wait()


def _gather(p1, p2, ei, kslice):
    mesh = plsc.VectorSubcoreMesh(core_axis_name="c", subcore_axis_name="s")

    def wrapped(p1_hbm, p2_hbm, ei_hbm, g_hbm, *scratch):
        idx1, idx2 = scratch[0], scratch[1]
        rows_a = scratch[2:2 + _NBUF]
        rows_b = scratch[2 + _NBUF:2 + 2 * _NBUF]
        gsems = scratch[2 + 2 * _NBUF:2 + 3 * _NBUF]
        wsem = scratch[2 + 3 * _NBUF]
        _gather_body(kslice, p1_hbm, p2_hbm, ei_hbm, g_hbm,
                     idx1, idx2, rows_a, rows_b, gsems, wsem)

    fn = functools.partial(
        pl.kernel,
        mesh=mesh,
        out_type=jax.ShapeDtypeStruct((_S, HIDDEN), jnp.float32),
        scratch_types=(
            [pltpu.VMEM((_BPW,), jnp.int32)] * 2
            + [pltpu.VMEM((_C, HIDDEN), jnp.float32)] * (2 * _NBUF)
            + [pltpu.SemaphoreType.DMA] * (_NBUF + 1)
        ),
    )(wrapped)
    return fn(p1, p2, ei)


_EB = 6400
_SLICE_BLOCKS = _S // _EB


def _we_body(wsp_ref, w3_ref, we_ref):
    we_ref[...] = jnp.dot(wsp_ref[...], w3_ref[...],
                          preferred_element_type=jnp.float32)


def _fold_we(wsp, w3):
    return pl.pallas_call(
        _we_body,
        out_shape=jax.ShapeDtypeStruct((D_EDGE, HIDDEN), jnp.float32),
    )(wsp, w3)


def _combine_body_first(g_ref, ea_ref, we_ref, b_ref, out_ref):
    e = lax.dot_general(ea_ref[...], we_ref[...],
                        dimension_numbers=(((0,), (0,)), ((), ())),
                        preferred_element_type=jnp.float32)
    acc = g_ref[...] + e + b_ref[...]
    out_ref[...] = jnp.maximum(acc, 0.0)


def _combine_body(prev_ref, g_ref, ea_ref, we_ref, b_ref, out_ref):
    del prev_ref
    _combine_body_first(g_ref, ea_ref, we_ref, b_ref, out_ref)


def _combine_slice(prev, g, ea_t, we, b, k):
    blk0 = k * _SLICE_BLOCKS
    data_specs = [
        pl.BlockSpec((_EB, HIDDEN), lambda i: (i, 0)),
        pl.BlockSpec((D_EDGE, _EB), lambda i, _b=blk0: (0, i + _b)),
        pl.BlockSpec((D_EDGE, HIDDEN), lambda i: (0, 0)),
        pl.BlockSpec((1, HIDDEN), lambda i: (0, 0)),
    ]
    out_spec = pl.BlockSpec((_EB, HIDDEN), lambda i, _b=blk0: (i + _b, 0))
    out_shape = jax.ShapeDtypeStruct((N_EDGES, HIDDEN), jnp.float32)
    if k == 0:
        return pl.pallas_call(
            _combine_body_first,
            grid=(_SLICE_BLOCKS,),
            in_specs=data_specs,
            out_specs=out_spec,
            out_shape=out_shape,
        )(g, ea_t, we, b)
    return pl.pallas_call(
        _combine_body,
        grid=(_SLICE_BLOCKS,),
        in_specs=[pl.BlockSpec((8, HIDDEN), lambda i: (0, 0))] + data_specs,
        out_specs=out_spec,
        out_shape=out_shape,
        input_output_aliases={0: 0},
    )(prev, g, ea_t, we, b)


def kernel(node_feat, edge_index, edge_attr, W_spatial, W_fc, b_fc):
    ei = edge_index.astype(jnp.int32).reshape(2 * N_EDGES)
    ea_t = edge_attr.T
    w1 = W_fc[:D_NODE]
    w2 = W_fc[D_NODE:2 * D_NODE]
    w3 = W_fc[2 * D_NODE:]
    b = b_fc.reshape(1, HIDDEN)
    p1, p2 = _node_proj(node_feat, w1, w2)
    we = _fold_we(W_spatial, w3)
    gs = [_gather(p1, p2, ei, k) for k in range(_K)]
    out = None
    for k in range(_K):
        out = _combine_slice(out, gs[k], ea_t, we, b, k)
    return out


# speedup vs baseline: 1.0506x; 1.0506x over previous
import functools

import jax
import jax.numpy as jnp
from jax import lax
from jax.experimental import pallas as pl
from jax.experimental.pallas import tpu as pltpu
from jax.experimental.pallas import tpu_sc as plsc

N_NODES = 10000
N_EDGES = 320000
D_NODE = 128
D_EDGE = 16
EMBED = 32
HIDDEN = 128

_K = 5
_S = N_EDGES // _K


_RB = 2000


def _proj_body(nf_ref, w1_ref, w2_ref, p1_ref, p2_ref):
    x = nf_ref[...]
    p1_ref[...] = jnp.dot(x, w1_ref[...], preferred_element_type=jnp.float32)
    p2_ref[...] = jnp.dot(x, w2_ref[...], preferred_element_type=jnp.float32)


def _node_proj(node_feat, w1, w2):
    n_blocks = N_NODES // _RB
    return pl.pallas_call(
        _proj_body,
        grid=(n_blocks,),
        in_specs=[
            pl.BlockSpec((_RB, D_NODE), lambda i: (i, 0)),
            pl.BlockSpec((D_NODE, HIDDEN), lambda i: (0, 0)),
            pl.BlockSpec((D_NODE, HIDDEN), lambda i: (0, 0)),
        ],
        out_specs=[
            pl.BlockSpec((_RB, HIDDEN), lambda i: (i, 0)),
            pl.BlockSpec((_RB, HIDDEN), lambda i: (i, 0)),
        ],
        out_shape=[
            jax.ShapeDtypeStruct((N_NODES, HIDDEN), jnp.float32),
            jax.ShapeDtypeStruct((N_NODES, HIDDEN), jnp.float32),
        ],
    )(node_feat, w1, w2)


_NC = 2
_NS = 16
_NW = _NC * _NS
_BPW = _S // _NW
_C = 16
_NCHUNK = _BPW // _C
_NBUF = 25


def _accum_rows(rows_a, rows_b):
    def row_body(r, carry):
        for c in range(HIDDEN // 16):
            sl = pl.ds(c * 16, 16)
            plsc.addupdate(rows_a.at[r, sl], rows_b[r, sl])
        return carry
    lax.fori_loop(0, _C, row_body, 0)


def _gather_body(kslice, p1_hbm, p2_hbm, ei_hbm, g_hbm,
                 idx1, idx2, rows_a, rows_b, gsems, wsem):
    wid = lax.axis_index("s") * _NC + lax.axis_index("c")
    base = pl.multiple_of(kslice * _S + wid * _BPW, 8)
    obase0 = pl.multiple_of(wid * _BPW, 8)
    pltpu.sync_copy(ei_hbm.at[pl.ds(base, _BPW)], idx1)
    pltpu.sync_copy(ei_hbm.at[pl.ds(N_EDGES + base, _BPW)], idx2)

    def fire(off, s):
        cp1 = pltpu.async_copy(p1_hbm.at[idx1.at[pl.ds(off, _C)]],
                               rows_a[s], gsems[s])
        cp2 = pltpu.async_copy(p2_hbm.at[idx2.at[pl.ds(off, _C)]],
                               rows_b[s], gsems[s])
        return cp1, cp2

    def drain(off, s, cps):
        cps[0].wait()
        cps[1].wait()
        _accum_rows(rows_a[s], rows_b[s])
        obase = pl.multiple_of(obase0 + off, 8)
        return pltpu.async_copy(rows_a[s], g_hbm.at[pl.ds(obase, _C)], wsem)

    def body(pi, carry):
        offs = [pl.multiple_of(pi * (_NBUF * _C) + s * _C, 8)
                for s in range(_NBUF)]
        cps = [fire(offs[s], s) for s in range(_NBUF)]
        ws = [drain(offs[s], s, cps[s]) for s in range(_NBUF)]
        for w in ws:
            w.wait()
        return carry

    lax.fori_loop(0, _NCHUNK // _NBUF, body, 0)
    for t in range(_NCHUNK - (_NCHUNK // _NBUF) * _NBUF):
        off = pl.multiple_of(((_NCHUNK // _NBUF) * _NBUF + t) * _C, 8)
        drain(off, t, fire(off, t)).wait()


def _gather(p1, p2, ei, kslice):
    mesh = plsc.VectorSubcoreMesh(core_axis_name="c", subcore_axis_name="s")

    def wrapped(p1_hbm, p2_hbm, ei_hbm, g_hbm, *scratch):
        idx1, idx2 = scratch[0], scratch[1]
        rows_a = scratch[2:2 + _NBUF]
        rows_b = scratch[2 + _NBUF:2 + 2 * _NBUF]
        gsems = scratch[2 + 2 * _NBUF:2 + 3 * _NBUF]
        wsem = scratch[2 + 3 * _NBUF]
        _gather_body(kslice, p1_hbm, p2_hbm, ei_hbm, g_hbm,
                     idx1, idx2, rows_a, rows_b, gsems, wsem)

    fn = functools.partial(
        pl.kernel,
        mesh=mesh,
        out_type=jax.ShapeDtypeStruct((_S, HIDDEN), jnp.float32),
        scratch_types=(
            [pltpu.VMEM((_BPW,), jnp.int32)] * 2
            + [pltpu.VMEM((_C, HIDDEN), jnp.float32)] * (2 * _NBUF)
            + [pltpu.SemaphoreType.DMA] * (_NBUF + 1)
        ),
    )(wrapped)
    return fn(p1, p2, ei)


_EB = 6400
_SLICE_BLOCKS = _S // _EB


def _we_body(wsp_ref, w3_ref, we_ref):
    we_ref[...] = jnp.dot(wsp_ref[...], w3_ref[...],
                          preferred_element_type=jnp.float32)


def _fold_we(wsp, w3):
    return pl.pallas_call(
        _we_body,
        out_shape=jax.ShapeDtypeStruct((D_EDGE, HIDDEN), jnp.float32),
    )(wsp, w3)


def _combine_body_first(g_ref, ea_ref, we_ref, b_ref, out_ref):
    e = lax.dot_general(ea_ref[...], we_ref[...],
                        dimension_numbers=(((0,), (0,)), ((), ())),
                        preferred_element_type=jnp.float32)
    acc = g_ref[...] + e + b_ref[...]
    out_ref[...] = jnp.maximum(acc, 0.0)


def _combine_body(prev_ref, g_ref, ea_ref, we_ref, b_ref, out_ref):
    del prev_ref
    _combine_body_first(g_ref, ea_ref, we_ref, b_ref, out_ref)


def _combine_slice(prev, g, ea_t, we, b, k):
    blk0 = k * _SLICE_BLOCKS
    data_specs = [
        pl.BlockSpec((_EB, HIDDEN), lambda i: (i, 0)),
        pl.BlockSpec((D_EDGE, _EB), lambda i, _b=blk0: (0, i + _b)),
        pl.BlockSpec((D_EDGE, HIDDEN), lambda i: (0, 0)),
        pl.BlockSpec((1, HIDDEN), lambda i: (0, 0)),
    ]
    out_spec = pl.BlockSpec((_EB, HIDDEN), lambda i, _b=blk0: (i + _b, 0))
    out_shape = jax.ShapeDtypeStruct((N_EDGES, HIDDEN), jnp.float32)
    if k == 0:
        return pl.pallas_call(
            _combine_body_first,
            grid=(_SLICE_BLOCKS,),
            in_specs=data_specs,
            out_specs=out_spec,
            out_shape=out_shape,
        )(g, ea_t, we, b)
    return pl.pallas_call(
        _combine_body,
        grid=(_SLICE_BLOCKS,),
        in_specs=[pl.BlockSpec((8, HIDDEN), lambda i: (0, 0))] + data_specs,
        out_specs=out_spec,
        out_shape=out_shape,
        input_output_aliases={0: 0},
    )(prev, g, ea_t, we, b)


def kernel(node_feat, edge_index, edge_attr, W_spatial, W_fc, b_fc):
    ei = edge_index.astype(jnp.int32).reshape(2 * N_EDGES)
    ea_t = edge_attr.T
    w1 = W_fc[:D_NODE]
    w2 = W_fc[D_NODE:2 * D_NODE]
    w3 = W_fc[2 * D_NODE:]
    b = b_fc.reshape(1, HIDDEN)
    p1, p2 = _node_proj(node_feat, w1, w2)
    we = _fold_we(W_spatial, w3)
    gs = [_gather(p1, p2, ei, k) for k in range(_K)]
    out = None
    for k in range(_K):
        out = _combine_slice(out, gs[k], ea_t, we, b, k)
    return out
